# baseline (speedup 1.0000x reference)
import functools

import jax
import jax.numpy as jnp
from jax import lax
from jax.experimental import pallas as pl
from jax.experimental.pallas import tpu as pltpu
from jax.experimental.pallas import tpu_sc as plsc

_PULL_W = 0.25
_PUSH_W = 0.25
_MARGIN = 2.0

_B, _C, _H, _W = 8, 16, 256, 256
_HW = _H * _W
_N = 20
_K = 64

_NC, _NS, _L = 2, 16, 16
_NW = _NC * _NS
_CPW = (_B * _C) // _NW
_TPB = _NW // _B
_QR = _H // 4
_NBUF = 6
_OW = 3 * _L


def _sc_gather_reduce(pred2, inds):
  mesh = plsc.VectorSubcoreMesh(core_axis_name="c", subcore_axis_name="s")

  @functools.partial(
      pl.kernel,
      out_type=jax.ShapeDtypeStruct((_NW, _OW), jnp.float32),
      mesh=mesh,
      compiler_params=pltpu.CompilerParams(needs_layout_passes=False),
      scratch_types=[
          pltpu.VMEM((_N, _K), jnp.int32),
          [pltpu.VMEM((_QR, _W), jnp.float32) for _ in range(_NBUF)],
          pltpu.VMEM((_N * _L,), jnp.float32),
          pltpu.VMEM((_OW,), jnp.float32),
          pltpu.SemaphoreType.DMA,
      ],
  )
  def k(pred_hbm, inds_hbm, out_hbm, ik_v, bufs, y_v, o_v, sem):
    wid = lax.axis_index("s") * _NC + lax.axis_index("c")
    b = wid // _TPB
    c0 = (wid % _TPB) * _CPW
    lane = lax.iota(jnp.int32, _L)

    pltpu.sync_copy(inds_hbm.at[b], ik_v)

    copies = {}

    def fire(qi):
      j, qq = divmod(qi, 4)
      rbase = (b * _C + c0 + j) * _H + qq * _QR
      copies[qi] = pltpu.async_copy(
          pred_hbm.at[pl.ds(rbase, _QR)], bufs[qi % _NBUF], sem)

    for qi in range(_NBUF):
      fire(qi)

    carry = (jnp.zeros((_L,), jnp.float32), jnp.zeros((_L,), jnp.float32),
             jnp.zeros((_L,), jnp.float32), jnp.float32(0.0))
    for j in range(_CPW):
      for half in range(2):
        q0 = 4 * j + 2 * half
        copies.pop(q0).wait()
        copies.pop(q0 + 1).wait()
        blo, bhi = bufs[q0 % _NBUF], bufs[(q0 + 1) % _NBUF]

        def nbody(n, c, blo=blo, bhi=bhi, half=half):
          sq_acc, sv0, sv1, b2 = c
          y = jnp.zeros((_L,), jnp.float32)
          for q in range(_K // _L):
            p = ik_v[n, pl.ds(q * _L, _L)]
            row = lax.shift_right_logical(p, 8)
            qsel = lax.bitwise_and(lax.shift_right_logical(p, 14), 1)
            hsel = lax.shift_right_logical(p, 15)
            rowm = lax.bitwise_and(row, _QR - 1)
            col = lax.bitwise_and(p, _W - 1)
            glo = plsc.load_gather(blo, [rowm, col])
            ghi = plsc.load_gather(bhi, [rowm, col])
            g = jnp.where(qsel == 0, glo, ghi)
            v = jnp.where(hsel == half, g, 0.0)
            y = y + v
            sq_acc = sq_acc + v * v
          if half == 0:
            y_v[pl.ds(n * _L, _L)] = y
          else:
            s_c = jnp.sum(y_v[pl.ds(n * _L, _L)] + y)
            b2 = b2 + s_c * s_c
            sv0 = sv0 + jnp.where(lane == n, s_c, 0.0)
            sv1 = sv1 + jnp.where(lane == n - _L, s_c, 0.0)
          return (sq_acc, sv0, sv1, b2)

        carry = lax.fori_loop(0, _N, nbody, carry)
        for qi in range(q0 + _NBUF, min(q0 + _NBUF + 2, 4 * _CPW)):
          fire(qi)

    sq_acc, sv0, sv1, b2 = carry
    a = jnp.sum(sq_acc)
    o_v[pl.ds(0, _L)] = sv0
    o_v[pl.ds(_L, _L)] = sv1
    o_v[pl.ds(2 * _L, _L)] = (jnp.where(lane == 0, a, 0.0) +
                              jnp.where(lane == 1, b2, 0.0))
    pltpu.sync_copy(o_v, out_hbm.at[wid])

  return k(pred2, inds)


def _tc_finish(parts):
  def body(x_ref, o_ref, o2_ref):
    x = x_ref[...]
    a = jnp.sum(x[:, 2 * _L:2 * _L + 1])
    b2 = jnp.sum(x[:, 2 * _L + 1:2 * _L + 2])
    pull_all = _PULL_W * (a * (1.0 / _K) - b2 * (1.0 / (_K * _K)))

    sv = jnp.sum(x.reshape(_B, _TPB, _OW), axis=1)[:, :2 * _L]
    s = sv * (1.0 / _K)
    lane2 = lax.broadcasted_iota(jnp.int32, (_B, 2 * _L, 2 * _L), 1)
    lane3 = lax.broadcasted_iota(jnp.int32, (_B, 2 * _L, 2 * _L), 2)
    diff = s[:, :, None] - s[:, None, :]
    m = jnp.maximum(_MARGIN - jnp.abs(diff), 0.0)
    valid = (lane2 < _N) & (lane3 < _N) & (lane2 != lane3)
    m = jnp.where(valid, m, 0.0)
    push_all = _PUSH_W * jnp.sum(m) / (_N * (_N - 1))

    o_ref[...] = jnp.zeros((1, 1), jnp.float32) + pull_all
    o2_ref[...] = jnp.zeros((1, 1), jnp.float32) + push_all

  return pl.pallas_call(
      body,
      out_shape=[
          jax.ShapeDtypeStruct((1, 1), jnp.float32),
          jax.ShapeDtypeStruct((1, 1), jnp.float32),
      ],
  )(parts)


@jax.jit
def kernel(pred, inds):
  pred2 = pred.reshape(_B * _C * _H, _W)
  parts = _sc_gather_reduce(pred2, inds)
  pull_all, push_all = _tc_finish(parts)
  return (pull_all.reshape(()), push_all.reshape(()))
